# baseline (speedup 1.0000x reference)
import functools
import math

import jax
import jax.numpy as jnp
from jax import lax
from jax.experimental import pallas as pl
from jax.experimental.pallas import tpu as pltpu
from jax.experimental.pallas import tpu_sc as plsc

D_EMBED = 64
SCALE = math.sqrt(D_EMBED)

NC, NS = 2, 16
NW = NC * NS
XR_PER_CHUNK = 2


def _make_kernel(R, S):
    assert R % (NW * XR_PER_CHUNK) == 0
    xr_per_w = R // NW
    n_chunks = xr_per_w // XR_PER_CHUNK
    assert n_chunks % 2 == 0
    mesh = plsc.VectorSubcoreMesh(
        core_axis_name="c", subcore_axis_name="s",
        num_cores=NC, num_subcores=NS)

    @functools.partial(
        pl.kernel,
        out_type=jax.ShapeDtypeStruct((R, S, D_EMBED), jnp.float32),
        mesh=mesh,
        scratch_types=[
            pltpu.VMEM((xr_per_w, S), jnp.int32),
            pltpu.VMEM((XR_PER_CHUNK, S, D_EMBED), jnp.float32),
            pltpu.VMEM((XR_PER_CHUNK, S, D_EMBED), jnp.float32),
            pltpu.SemaphoreType.DMA((2,)),
            pltpu.SemaphoreType.DMA((2,)),
        ],
        compiler_params=pltpu.CompilerParams(use_tc_tiling_on_sc=False),
    )
    def k(x_hbm, table_hbm, out_hbm, idx_v, rows0, rows1, gsem, ssem):
        wid = lax.axis_index("s") * NC + lax.axis_index("c")
        xr0 = wid * xr_per_w
        pltpu.sync_copy(x_hbm.at[pl.ds(xr0, xr_per_w)], idx_v)
        bufs = (rows0, rows1)

        def gather_start(g, b):
            for i in range(XR_PER_CHUNK):
                pltpu.async_copy(
                    table_hbm.at[idx_v.at[XR_PER_CHUNK * g + i]],
                    bufs[b].at[i], gsem.at[b])

        def gather_wait(b):
            for i in range(XR_PER_CHUNK):
                pltpu.make_async_copy(
                    table_hbm.at[idx_v.at[0]], bufs[b].at[i],
                    gsem.at[b]).wait()

        def store_start(g, b):
            pltpu.async_copy(
                bufs[b], out_hbm.at[pl.ds(xr0 + XR_PER_CHUNK * g, XR_PER_CHUNK)],
                ssem.at[b])

        def store_wait(b):
            pltpu.make_async_copy(
                bufs[b], out_hbm.at[pl.ds(xr0, XR_PER_CHUNK)], ssem.at[b]).wait()

        def scale(b):
            buf = bufs[b]

            def body(r, c):
                for i in range(XR_PER_CHUNK):
                    for p in range(D_EMBED // 16):
                        sl = (i, r, pl.ds(p * 16, 16))
                        buf[sl] = buf[sl] * SCALE
                return c
            lax.fori_loop(0, S, body, 0)

        gather_start(0, 0)

        def pair(t, c):
            for ph in range(2):
                g = 2 * t + ph
                b, nb = ph, 1 - ph

                @pl.when(jnp.logical_and(g >= 1, g + 1 < n_chunks))
                def _():
                    store_wait(nb)

                @pl.when(g + 1 < n_chunks)
                def _():
                    gather_start(g + 1, nb)

                gather_wait(b)
                scale(b)
                store_start(g, b)
            return c
        lax.fori_loop(0, n_chunks // 2, pair, 0)
        store_wait(0)
        store_wait(1)

    return k


def kernel(x, table):
    R, S = x.shape
    return _make_kernel(R, S)(x.astype(jnp.int32), table)


# speedup vs baseline: 1.0202x; 1.0202x over previous
import functools
import math

import jax
import jax.numpy as jnp
from jax import lax
from jax.experimental import pallas as pl
from jax.experimental.pallas import tpu as pltpu
from jax.experimental.pallas import tpu_sc as plsc

D_EMBED = 64
SCALE = math.sqrt(D_EMBED)

NC, NS = 2, 16
NW = NC * NS
CH = 128
PK = CH // 2


def _make_kernel(B):
    assert B % (NW * CH) == 0
    n_chunks = B // (NW * CH)
    assert n_chunks % 2 == 0
    b_per_w = B // NW
    mesh = plsc.VectorSubcoreMesh(
        core_axis_name="c", subcore_axis_name="s",
        num_cores=NC, num_subcores=NS)

    @functools.partial(
        pl.kernel,
        out_type=jax.ShapeDtypeStruct((B // 2, 2 * D_EMBED), jnp.float32),
        mesh=mesh,
        scratch_types=[
            pltpu.VMEM((b_per_w,), jnp.int32),
            pltpu.VMEM((CH, D_EMBED), jnp.float32),
            pltpu.VMEM((CH, D_EMBED), jnp.float32),
            pltpu.VMEM((PK, 2 * D_EMBED), jnp.float32),
            pltpu.VMEM((PK, 2 * D_EMBED), jnp.float32),
            pltpu.SemaphoreType.DMA((2,)),
            pltpu.SemaphoreType.DMA((2,)),
        ],
        compiler_params=pltpu.CompilerParams(use_tc_tiling_on_sc=False),
    )
    def k(x_hbm, table_hbm, out_hbm, idx_v, gb0, gb1, ob0, ob1, gsem, ssem):
        wid = lax.axis_index("s") * NC + lax.axis_index("c")
        pltpu.sync_copy(x_hbm.at[pl.ds(wid * b_per_w, b_per_w)], idx_v)
        gbufs = (gb0, gb1)
        obufs = (ob0, ob1)

        def gather_start(g, b):
            pltpu.async_copy(
                table_hbm.at[idx_v.at[pl.ds(g * CH, CH)]], gbufs[b],
                gsem.at[b])

        def gather_wait(b):
            pltpu.make_async_copy(
                table_hbm.at[idx_v.at[pl.ds(0, CH)]], gbufs[b],
                gsem.at[b]).wait()

        def store_start(g, b):
            pltpu.async_copy(
                obufs[b],
                out_hbm.at[pl.ds((wid * n_chunks + g) * PK, PK)], ssem.at[b])

        def store_wait(b):
            pltpu.make_async_copy(
                obufs[b], out_hbm.at[pl.ds(0, PK)], ssem.at[b]).wait()

        def scale_pack(b):
            gb, ob = gbufs[b], obufs[b]

            def body(r, c):
                for i in range(2):
                    for p in range(D_EMBED // 16):
                        ob[r, pl.ds(i * D_EMBED + p * 16, 16)] = (
                            gb[2 * r + i, pl.ds(p * 16, 16)] * SCALE)
                return c
            lax.fori_loop(0, PK, body, 0)

        gather_start(0, 0)

        def pair(t, c):
            for ph in range(2):
                g = 2 * t + ph
                b, nb = ph, 1 - ph

                @pl.when(jnp.logical_and(g >= 1, g + 1 < n_chunks))
                def _():
                    store_wait(nb)

                @pl.when(g + 1 < n_chunks)
                def _():
                    gather_start(g + 1, nb)

                gather_wait(b)
                scale_pack(b)
                store_start(g, b)
            return c
        lax.fori_loop(0, n_chunks // 2, pair, 0)
        store_wait(0)
        store_wait(1)

    return k


def kernel(x, table):
    R, S = x.shape
    B = R * S
    x1d = x.reshape(B).astype(jnp.int32)
    out = _make_kernel(B)(x1d, table)
    return out.reshape(R, S, D_EMBED)
